# baseline (speedup 1.0000x reference)
import jax
import jax.numpy as jnp
from jax import lax
from jax.experimental import pallas as pl
from jax.experimental.pallas import tpu as pltpu
from jax.experimental.pallas import tpu_sc as plsc

N_USERS = 2000
N_ENTITIES = 8000
N_NODES = N_USERS + N_ENTITIES
N_EDGES = 320000
D = 128
EPS = 1e-12

NC = 2
NS = 16
NW = NC * NS
C = 80
NCHUNK = 125
NE_W = NCHUNK * C
E_PAD = NW * NE_W
N_PAD = 10240
ROWS_PER_TILE = N_PAD // NS
NRB = 4
NEB = 6
SUPER = 12


def _sc_spmm_body(x_hbm, edges_hbm, vals_hbm, out_hbm,
                  acc, eb0, eb1, eb2, eb3, eb4, eb5,
                  vb0, vb1, vb2, vb3, vb4, vb5,
                  rows0, rows1, rows2, rows3,
                  e0, e1, e2, e3, e4, e5, g0, g1, g2, g3, s0, s1, s2, s3):
    c = lax.axis_index("c")
    s = lax.axis_index("s")
    wid = s * NC + c

    ebs = [eb0, eb1, eb2, eb3, eb4, eb5]
    vbs = [vb0, vb1, vb2, vb3, vb4, vb5]
    rws = [rows0, rows1, rows2, rows3]
    ess = [e0, e1, e2, e3, e4, e5]
    gss = [g0, g1, g2, g3]
    sss = [s0, s1, s2, s3]

    def _zero_row(i, _):
        for f in range(D // 16):
            rows0[i, pl.ds(f * 16, 16)] = jnp.zeros((16,), jnp.float32)
        return 0
    lax.fori_loop(0, C, _zero_row, 0)
    nz = ROWS_PER_TILE // C
    rem = ROWS_PER_TILE % C
    for z in range(nz):
        pltpu.async_copy(rows0, acc.at[pl.ds(s * ROWS_PER_TILE + z * C, C)], g0)
    if rem:
        pltpu.async_copy(
            rows0.at[pl.ds(0, rem)],
            acc.at[pl.ds(s * ROWS_PER_TILE + nz * C, rem)], g0)
    for z in range(nz):
        pltpu.make_async_copy(
            rows0, acc.at[pl.ds(s * ROWS_PER_TILE + z * C, C)], g0).wait()
    if rem:
        pltpu.make_async_copy(
            rows0.at[pl.ds(0, rem)],
            acc.at[pl.ds(s * ROWS_PER_TILE + nz * C, rem)], g0).wait()
    plsc.subcore_barrier()

    def start_eload(k, j):
        pltpu.async_copy(edges_hbm.at[wid, k], ebs[j], ess[j])
        pltpu.async_copy(vals_hbm.at[wid, k, 0], vbs[j], ess[j])

    def wait_eload(j):
        pltpu.make_async_copy(edges_hbm.at[wid, 0], ebs[j], ess[j]).wait()
        pltpu.make_async_copy(vals_hbm.at[wid, 0, 0], vbs[j], ess[j]).wait()

    def start_gather(j, r):
        pltpu.async_copy(x_hbm.at[ebs[j].at[0]], rws[r], gss[r])

    def wait_gather(j, r):
        pltpu.make_async_copy(x_hbm.at[ebs[j].at[0]], rws[r], gss[r]).wait()

    def start_scatter(j, r):
        pltpu.async_copy(rws[r], acc.at[ebs[j].at[1]], sss[r], add=True)

    def wait_scatter(j, r):
        pltpu.make_async_copy(rws[r], acc.at[ebs[j].at[1]], sss[r]).wait()

    def scale(j, r):
        vb = vbs[j]
        buf = rws[r]

        def grp(g, _):
            vv = vb[pl.ds(g * 16, 16)]
            dn = lax.GatherDimensionNumbers(
                offset_dims=(), collapsed_slice_dims=(0,), start_index_map=(0,))
            for i in range(16):
                bv = lax.gather(
                    vv, jnp.full((16, 1), i, jnp.int32), dn, (1,),
                    mode=lax.GatherScatterMode.PROMISE_IN_BOUNDS)
                e = g * 16 + i
                for f in range(D // 16):
                    buf[e, pl.ds(f * 16, 16)] = buf[e, pl.ds(f * 16, 16)] * bv
            return 0
        lax.fori_loop(0, C // 16, grp, 0)

    MS = (NCHUNK - 5) // SUPER

    start_eload(0, 0)
    start_eload(1, 1)
    start_eload(2, 2)
    start_eload(3, 3)
    wait_eload(0)
    start_gather(0, 0)
    wait_eload(1)
    start_gather(1, 1)

    def body(mm, _):
        for j in range(SUPER):
            r = j % NRB
            je = j % NEB
            wait_gather(je, r)
            scale(je, r)
            start_scatter(je, r)

            if j <= 1:
                @pl.when(mm > 0)
                def _():
                    wait_scatter((je - 2) % NEB, (r - 2) % NRB)
            else:
                wait_scatter((je - 2) % NEB, (r - 2) % NRB)

            start_eload(SUPER * mm + j + 4, (je + 4) % NEB)
            wait_eload((je + 2) % NEB)
            start_gather((je + 2) % NEB, (r + 2) % NRB)
        return 0

    lax.fori_loop(0, MS, body, 0)
    for k in range(SUPER * MS, NCHUNK):
        r = k % NRB
        je = k % NEB
        wait_gather(je, r)
        scale(je, r)
        start_scatter(je, r)
        wait_scatter((je - 2) % NEB, (r - 2) % NRB)
        if k + 4 < NCHUNK:
            start_eload(k + 4, (je + 4) % NEB)
        if k + 2 < NCHUNK:
            wait_eload((je + 2) % NEB)
            start_gather((je + 2) % NEB, (r + 2) % NRB)
    wait_scatter((NCHUNK - 2) % NEB, (NCHUNK - 2) % NRB)
    wait_scatter((NCHUNK - 1) % NEB, (NCHUNK - 1) % NRB)
    plsc.subcore_barrier()

    pltpu.sync_copy(acc.at[pl.ds(s * ROWS_PER_TILE, ROWS_PER_TILE)],
                    out_hbm.at[c, pl.ds(s * ROWS_PER_TILE, ROWS_PER_TILE)])


def _make_sc_spmm():
    mesh = plsc.VectorSubcoreMesh(core_axis_name="c", subcore_axis_name="s")
    return pl.kernel(
        _sc_spmm_body,
        out_type=jax.ShapeDtypeStruct((NC, N_PAD, D), jnp.float32),
        mesh=mesh,
        scratch_types=(
            [pltpu.VMEM_SHARED((N_PAD, D), jnp.float32)]
            + [pltpu.VMEM((2, C), jnp.int32) for _ in range(NEB)]
            + [pltpu.VMEM((C,), jnp.float32) for _ in range(NEB)]
            + [pltpu.VMEM((C, D), jnp.float32) for _ in range(NRB)]
            + [pltpu.SemaphoreType.DMA for _ in range(NEB + 2 * NRB)]
        ),
    )


_TC_ROWS = 2000
_NU_BLK = N_USERS // _TC_ROWS
_NI_BLK = N_ENTITIES // _TC_ROWS


def _l2n(x):
    nrm = jnp.sqrt(jnp.sum(x * x, axis=1, keepdims=True))
    return x / jnp.maximum(nrm, EPS)


def _dense(ego, parts, w):
    bi = ego * (parts[0] + parts[1])
    h = jnp.dot(bi, w, preferred_element_type=jnp.float32)
    return jnp.where(h > 0, h, h * 0.2)


def _tc_h_body(ego_ref, parts_ref, w_ref, h_ref):
    h_ref[...] = _dense(ego_ref[...], parts_ref, w_ref[...])


def _tc_hn_body(ego_ref, parts_ref, w_ref, h_ref, n_ref):
    ego = ego_ref[...]
    h_ref[...] = _dense(ego, parts_ref, w_ref[...])
    n_ref[...] = _l2n(ego)


_layer_in_specs = [
    pl.BlockSpec((_TC_ROWS, D), lambda i: (i, 0)),
    pl.BlockSpec((NC, _TC_ROWS, D), lambda i: (0, i, 0)),
    pl.BlockSpec((D, D), lambda i: (0, 0)),
]
_row_out_spec = pl.BlockSpec((_TC_ROWS, D), lambda i: (i, 0))

_tc_h = pl.pallas_call(
    _tc_h_body,
    grid=(N_NODES // _TC_ROWS,),
    in_specs=_layer_in_specs,
    out_specs=_row_out_spec,
    out_shape=jax.ShapeDtypeStruct((N_NODES, D), jnp.float32),
)

_tc_hn = pl.pallas_call(
    _tc_hn_body,
    grid=(N_NODES // _TC_ROWS,),
    in_specs=_layer_in_specs,
    out_specs=[_row_out_spec, _row_out_spec],
    out_shape=[
        jax.ShapeDtypeStruct((N_NODES, D), jnp.float32),
        jax.ShapeDtypeStruct((N_NODES, D), jnp.float32),
    ],
)


def _asm_body(e_ref, n1_ref, n2_ref, h3_ref, u_ref, i_ref):
    i = pl.program_id(0)
    n3 = _l2n(h3_ref[...])
    cols = (e_ref[...], n1_ref[...], n2_ref[...], n3)

    @pl.when(i == 0)
    def _():
        for t in range(4):
            u_ref[:, pl.ds(t * D, D)] = cols[t]

    @pl.when(i > 0)
    def _():
        for t in range(4):
            i_ref[:, pl.ds(t * D, D)] = cols[t]


_asm = pl.pallas_call(
    _asm_body,
    grid=(N_NODES // _TC_ROWS,),
    in_specs=[pl.BlockSpec((_TC_ROWS, D), lambda i: (i, 0))] * 4,
    out_specs=[
        pl.BlockSpec((N_USERS, 4 * D), lambda i: (0, 0)),
        pl.BlockSpec((_TC_ROWS, 4 * D), lambda i: (jnp.maximum(i - 1, 0), 0)),
    ],
    out_shape=[
        jax.ShapeDtypeStruct((N_USERS, 4 * D), jnp.float32),
        jax.ShapeDtypeStruct((N_ENTITIES, 4 * D), jnp.float32),
    ],
)


def kernel(user_embed, entity_embed, W0, W1, W2, edge_index, edge_vals):
    ego = jnp.concatenate([user_embed, entity_embed], axis=0)

    pad = E_PAD - N_EDGES
    fill = (jnp.arange(pad, dtype=jnp.int32) * 37) % N_NODES
    dst = jnp.concatenate([edge_index[0], fill]).reshape(NW, NCHUNK, C)
    src = jnp.concatenate([edge_index[1], fill]).reshape(NW, NCHUNK, C)
    val = jnp.concatenate(
        [edge_vals, jnp.zeros((pad,), jnp.float32)]).reshape(NW, NCHUNK, C)
    edges = jnp.stack([src, dst], axis=2)
    vals = val[:, :, None, :]

    sc_spmm = _make_sc_spmm()

    h1 = _tc_h(ego, sc_spmm(ego, edges, vals), W0)
    h2, n1 = _tc_hn(h1, sc_spmm(h1, edges, vals), W1)
    h3, n2 = _tc_hn(h2, sc_spmm(h2, edges, vals), W2)
    return _asm(ego, n1, n2, h3)


# speedup vs baseline: 1.0646x; 1.0646x over previous
import jax
import jax.numpy as jnp
from jax import lax
from jax.experimental import pallas as pl
from jax.experimental.pallas import tpu as pltpu
from jax.experimental.pallas import tpu_sc as plsc

N_USERS = 2000
N_ENTITIES = 8000
N_NODES = N_USERS + N_ENTITIES
N_EDGES = 320000
D = 128
EPS = 1e-12

NC = 2
NS = 16
NW = NC * NS
C = 112
NCHUNK = 90
NE_W = NCHUNK * C
E_PAD = NW * NE_W
N_PAD = 10240
ROWS_PER_TILE = N_PAD // NS
NRB = 3
NPB = 3
NPAIR = NCHUNK // 2
SUPER = 6


def _sc_spmm_body(x_hbm, edges_hbm, vals_hbm, out_hbm,
                  acc, eb0, eb1, eb2, vb0, vb1, vb2,
                  rows0, rows1, rows2,
                  e0, e1, e2, g0, g1, g2, s0, s1, s2):
    c = lax.axis_index("c")
    s = lax.axis_index("s")
    wid = s * NC + c

    ebs = [eb0, eb1, eb2]
    vbs = [vb0, vb1, vb2]
    rws = [rows0, rows1, rows2]
    ess = [e0, e1, e2]
    gss = [g0, g1, g2]
    sss = [s0, s1, s2]

    def _zero_row(i, _):
        for f in range(D // 16):
            rows0[i, pl.ds(f * 16, 16)] = jnp.zeros((16,), jnp.float32)
        return 0
    lax.fori_loop(0, C, _zero_row, 0)
    nz = ROWS_PER_TILE // C
    rem = ROWS_PER_TILE % C
    for z in range(nz):
        pltpu.async_copy(rows0, acc.at[pl.ds(s * ROWS_PER_TILE + z * C, C)], g0)
    if rem:
        pltpu.async_copy(
            rows0.at[pl.ds(0, rem)],
            acc.at[pl.ds(s * ROWS_PER_TILE + nz * C, rem)], g0)
    for z in range(nz):
        pltpu.make_async_copy(
            rows0, acc.at[pl.ds(s * ROWS_PER_TILE + z * C, C)], g0).wait()
    if rem:
        pltpu.make_async_copy(
            rows0.at[pl.ds(0, rem)],
            acc.at[pl.ds(s * ROWS_PER_TILE + nz * C, rem)], g0).wait()
    plsc.subcore_barrier()

    def start_pload(p, j):
        pltpu.async_copy(edges_hbm.at[wid, p], ebs[j], ess[j])
        pltpu.async_copy(vals_hbm.at[wid, p], vbs[j], ess[j])

    def wait_pload(j):
        pltpu.make_async_copy(edges_hbm.at[wid, 0], ebs[j], ess[j]).wait()
        pltpu.make_async_copy(vals_hbm.at[wid, 0], vbs[j], ess[j]).wait()

    def start_gather(j, h, r):
        pltpu.async_copy(x_hbm.at[ebs[j].at[2 * h]], rws[r], gss[r])

    def wait_gather(j, h, r):
        pltpu.make_async_copy(x_hbm.at[ebs[j].at[2 * h]], rws[r], gss[r]).wait()

    def start_scatter(j, h, r):
        pltpu.async_copy(rws[r], acc.at[ebs[j].at[2 * h + 1]], sss[r], add=True)

    def wait_scatter(j, h, r):
        pltpu.make_async_copy(rws[r], acc.at[ebs[j].at[2 * h + 1]], sss[r]).wait()

    def scale(j, h, r):
        vb = vbs[j]
        buf = rws[r]

        def grp(g, _):
            vv = vb[h, pl.ds(g * 16, 16)]
            dn = lax.GatherDimensionNumbers(
                offset_dims=(), collapsed_slice_dims=(0,), start_index_map=(0,))
            for i in range(16):
                bv = lax.gather(
                    vv, jnp.full((16, 1), i, jnp.int32), dn, (1,),
                    mode=lax.GatherScatterMode.PROMISE_IN_BOUNDS)
                e = g * 16 + i
                for f in range(D // 16):
                    buf[e, pl.ds(f * 16, 16)] = buf[e, pl.ds(f * 16, 16)] * bv
            return 0
        lax.fori_loop(0, C // 16, grp, 0)

    MS = (NCHUNK - 6) // SUPER

    start_pload(0, 0)
    start_pload(1, 1)
    wait_pload(0)
    start_gather(0, 0, 0)
    start_gather(0, 1, 1)

    def body(mm, _):
        for j in range(SUPER):
            r = j % NRB
            jp = j // 2
            h = j % 2
            wait_gather(jp, h, r)
            scale(jp, h, r)
            start_scatter(jp, h, r)

            if j == 0:
                @pl.when(mm > 0)
                def _():
                    wait_scatter((jp - 1) % NPB, 1, (r - 1) % NRB)
            else:
                wait_scatter((jp * 2 + h - 1) // 2 % NPB, (h + 1) % 2,
                             (r - 1) % NRB)

            if h == 0:
                start_pload(3 * mm + jp + 2, (jp + 2) % NPB)
                wait_pload((jp + 1) % NPB)
            start_gather((jp + 1) % NPB, h, (r + 2) % NRB)
        return 0

    lax.fori_loop(0, MS, body, 0)
    for k in range(SUPER * MS, NCHUNK):
        r = k % NRB
        p = k // 2
        jp = p % NPB
        h = k % 2
        wait_gather(jp, h, r)
        scale(jp, h, r)
        start_scatter(jp, h, r)
        wait_scatter(((k - 1) // 2) % NPB, (k - 1) % 2, (r - 1) % NRB)
        if h == 0 and p + 2 < NPAIR:
            start_pload(p + 2, (jp + 2) % NPB)
        if k + 2 < NCHUNK:
            if h == 0:
                wait_pload((jp + 1) % NPB)
            start_gather((jp + 1) % NPB, h, (r + 2) % NRB)
    wait_scatter((NCHUNK - 1) // 2 % NPB, (NCHUNK - 1) % 2, (NCHUNK - 1) % NRB)
    plsc.subcore_barrier()

    pltpu.sync_copy(acc.at[pl.ds(s * ROWS_PER_TILE, ROWS_PER_TILE)],
                    out_hbm.at[c, pl.ds(s * ROWS_PER_TILE, ROWS_PER_TILE)])


def _make_sc_spmm():
    mesh = plsc.VectorSubcoreMesh(core_axis_name="c", subcore_axis_name="s")
    return pl.kernel(
        _sc_spmm_body,
        out_type=jax.ShapeDtypeStruct((NC, N_PAD, D), jnp.float32),
        mesh=mesh,
        scratch_types=(
            [pltpu.VMEM_SHARED((N_PAD, D), jnp.float32)]
            + [pltpu.VMEM((4, C), jnp.int32) for _ in range(NPB)]
            + [pltpu.VMEM((2, C), jnp.float32) for _ in range(NPB)]
            + [pltpu.VMEM((C, D), jnp.float32) for _ in range(NRB)]
            + [pltpu.SemaphoreType.DMA for _ in range(NPB + 2 * NRB)]
        ),
    )


_TC_ROWS = 2000
_NU_BLK = N_USERS // _TC_ROWS
_NI_BLK = N_ENTITIES // _TC_ROWS


def _l2n(x):
    nrm = jnp.sqrt(jnp.sum(x * x, axis=1, keepdims=True))
    return x / jnp.maximum(nrm, EPS)


def _dense(ego, parts, w):
    bi = ego * (parts[0] + parts[1])
    h = jnp.dot(bi, w, preferred_element_type=jnp.float32)
    return jnp.where(h > 0, h, h * 0.2)


def _tc_h_body(ego_ref, parts_ref, w_ref, h_ref):
    h_ref[...] = _dense(ego_ref[...], parts_ref, w_ref[...])


def _tc_hn_body(ego_ref, parts_ref, w_ref, h_ref, n_ref):
    ego = ego_ref[...]
    h_ref[...] = _dense(ego, parts_ref, w_ref[...])
    n_ref[...] = _l2n(ego)


_layer_in_specs = [
    pl.BlockSpec((_TC_ROWS, D), lambda i: (i, 0)),
    pl.BlockSpec((NC, _TC_ROWS, D), lambda i: (0, i, 0)),
    pl.BlockSpec((D, D), lambda i: (0, 0)),
]
_row_out_spec = pl.BlockSpec((_TC_ROWS, D), lambda i: (i, 0))

_tc_h = pl.pallas_call(
    _tc_h_body,
    grid=(N_NODES // _TC_ROWS,),
    in_specs=_layer_in_specs,
    out_specs=_row_out_spec,
    out_shape=jax.ShapeDtypeStruct((N_NODES, D), jnp.float32),
)

_tc_hn = pl.pallas_call(
    _tc_hn_body,
    grid=(N_NODES // _TC_ROWS,),
    in_specs=_layer_in_specs,
    out_specs=[_row_out_spec, _row_out_spec],
    out_shape=[
        jax.ShapeDtypeStruct((N_NODES, D), jnp.float32),
        jax.ShapeDtypeStruct((N_NODES, D), jnp.float32),
    ],
)


def _asm_body(e_ref, n1_ref, n2_ref, h3_ref, u_ref, i_ref):
    i = pl.program_id(0)
    n3 = _l2n(h3_ref[...])
    cols = (e_ref[...], n1_ref[...], n2_ref[...], n3)

    @pl.when(i == 0)
    def _():
        for t in range(4):
            u_ref[:, pl.ds(t * D, D)] = cols[t]

    @pl.when(i > 0)
    def _():
        for t in range(4):
            i_ref[:, pl.ds(t * D, D)] = cols[t]


_asm = pl.pallas_call(
    _asm_body,
    grid=(N_NODES // _TC_ROWS,),
    in_specs=[pl.BlockSpec((_TC_ROWS, D), lambda i: (i, 0))] * 4,
    out_specs=[
        pl.BlockSpec((N_USERS, 4 * D), lambda i: (0, 0)),
        pl.BlockSpec((_TC_ROWS, 4 * D), lambda i: (jnp.maximum(i - 1, 0), 0)),
    ],
    out_shape=[
        jax.ShapeDtypeStruct((N_USERS, 4 * D), jnp.float32),
        jax.ShapeDtypeStruct((N_ENTITIES, 4 * D), jnp.float32),
    ],
)


def kernel(user_embed, entity_embed, W0, W1, W2, edge_index, edge_vals):
    ego = jnp.concatenate([user_embed, entity_embed], axis=0)

    pad = E_PAD - N_EDGES
    fill = (jnp.arange(pad, dtype=jnp.int32) * 37) % N_NODES
    dst = jnp.concatenate([edge_index[0], fill]).reshape(NW, NCHUNK, C)
    src = jnp.concatenate([edge_index[1], fill]).reshape(NW, NCHUNK, C)
    val = jnp.concatenate(
        [edge_vals, jnp.zeros((pad,), jnp.float32)]).reshape(NW, NCHUNK, C)
    srcp = src.reshape(NW, NPAIR, 2, C)
    dstp = dst.reshape(NW, NPAIR, 2, C)
    edges = jnp.stack(
        [srcp[:, :, 0], dstp[:, :, 0], srcp[:, :, 1], dstp[:, :, 1]],
        axis=2)
    vals = val.reshape(NW, NPAIR, 2, C)

    sc_spmm = _make_sc_spmm()

    h1 = _tc_h(ego, sc_spmm(ego, edges, vals), W0)
    h2, n1 = _tc_hn(h1, sc_spmm(h1, edges, vals), W1)
    h3, n2 = _tc_hn(h2, sc_spmm(h2, edges, vals), W2)
    return _asm(ego, n1, n2, h3)


# speedup vs baseline: 1.0649x; 1.0003x over previous
import jax
import jax.numpy as jnp
from jax import lax
from jax.experimental import pallas as pl
from jax.experimental.pallas import tpu as pltpu
from jax.experimental.pallas import tpu_sc as plsc

N_USERS = 2000
N_ENTITIES = 8000
N_NODES = N_USERS + N_ENTITIES
N_EDGES = 320000
D = 128
EPS = 1e-12

NC = 2
NS = 16
NW = NC * NS
C = 112
NCHUNK = 90
NE_W = NCHUNK * C
E_PAD = NW * NE_W
N_PAD = 10240
ROWS_PER_TILE = N_PAD // NS
NRB = 3
NPB = 3
NPAIR = NCHUNK // 2
SUPER = 6


def _sc_spmm_body(x_hbm, edges_hbm, vals_hbm, out_hbm,
                  acc, eb0, eb1, eb2, vb0, vb1, vb2,
                  rows0, rows1, rows2,
                  e0, e1, e2, g0, g1, g2, s0, s1, s2):
    c = lax.axis_index("c")
    s = lax.axis_index("s")
    wid = s * NC + c

    ebs = [eb0, eb1, eb2]
    vbs = [vb0, vb1, vb2]
    rws = [rows0, rows1, rows2]
    ess = [e0, e1, e2]
    gss = [g0, g1, g2]
    sss = [s0, s1, s2]

    def _zero_row(i, _):
        for f in range(D // 16):
            rows0[i, pl.ds(f * 16, 16)] = jnp.zeros((16,), jnp.float32)
        return 0
    lax.fori_loop(0, C, _zero_row, 0)
    nz = ROWS_PER_TILE // C
    rem = ROWS_PER_TILE % C
    @pl.when(s < NS)
    def _():
        for z in range(nz):
            pltpu.async_copy(rows0,
                             acc.at[pl.ds(s * ROWS_PER_TILE + z * C, C)], g0)
        if rem:
            pltpu.async_copy(
                rows0.at[pl.ds(0, rem)],
                acc.at[pl.ds(s * ROWS_PER_TILE + nz * C, rem)], g0)
    for z in range(nz):
        pltpu.make_async_copy(
            rows0, acc.at[pl.ds(s * ROWS_PER_TILE + z * C, C)], g0).wait()
    if rem:
        pltpu.make_async_copy(
            rows0.at[pl.ds(0, rem)],
            acc.at[pl.ds(s * ROWS_PER_TILE + nz * C, rem)], g0).wait()
    plsc.subcore_barrier()

    def start_pload(p, j):
        pltpu.async_copy(edges_hbm.at[wid, p], ebs[j], ess[j])
        pltpu.async_copy(vals_hbm.at[wid, p], vbs[j], ess[j])

    def wait_pload(j):
        pltpu.make_async_copy(edges_hbm.at[wid, 0], ebs[j], ess[j]).wait()
        pltpu.make_async_copy(vals_hbm.at[wid, 0], vbs[j], ess[j]).wait()

    def start_gather(j, h, r):
        pltpu.async_copy(x_hbm.at[ebs[j].at[2 * h]], rws[r], gss[r])

    def wait_gather(j, h, r):
        pltpu.make_async_copy(x_hbm.at[ebs[j].at[2 * h]], rws[r], gss[r]).wait()

    def start_scatter(j, h, r):
        @pl.when(s < NS)
        def _():
            pltpu.async_copy(rws[r], acc.at[ebs[j].at[2 * h + 1]],
                             sss[r], add=True)

    def wait_scatter(j, h, r):
        pltpu.make_async_copy(rws[r], acc.at[ebs[j].at[2 * h + 1]], sss[r]).wait()

    def scale(j, h, r):
        vb = vbs[j]
        buf = rws[r]

        def grp(g, _):
            vv = vb[h, pl.ds(g * 16, 16)]
            dn = lax.GatherDimensionNumbers(
                offset_dims=(), collapsed_slice_dims=(0,), start_index_map=(0,))
            for i in range(16):
                bv = lax.gather(
                    vv, jnp.full((16, 1), i, jnp.int32), dn, (1,),
                    mode=lax.GatherScatterMode.PROMISE_IN_BOUNDS)
                e = g * 16 + i
                for f in range(D // 16):
                    buf[e, pl.ds(f * 16, 16)] = buf[e, pl.ds(f * 16, 16)] * bv
            return 0
        lax.fori_loop(0, C // 16, grp, 0)

    MS = (NCHUNK - 6) // SUPER

    start_pload(0, 0)
    start_pload(1, 1)
    wait_pload(0)
    start_gather(0, 0, 0)
    start_gather(0, 1, 1)

    def body(mm, _):
        for j in range(SUPER):
            r = j % NRB
            jp = j // 2
            h = j % 2
            wait_gather(jp, h, r)
            scale(jp, h, r)
            start_scatter(jp, h, r)

            if j == 0:
                @pl.when(mm > 0)
                def _():
                    wait_scatter((jp - 1) % NPB, 1, (r - 1) % NRB)
            else:
                wait_scatter((jp * 2 + h - 1) // 2 % NPB, (h + 1) % 2,
                             (r - 1) % NRB)

            if h == 0:
                start_pload(3 * mm + jp + 2, (jp + 2) % NPB)
                wait_pload((jp + 1) % NPB)
            start_gather((jp + 1) % NPB, h, (r + 2) % NRB)
        return 0

    lax.fori_loop(0, MS, body, 0)
    for k in range(SUPER * MS, NCHUNK):
        r = k % NRB
        p = k // 2
        jp = p % NPB
        h = k % 2
        wait_gather(jp, h, r)
        scale(jp, h, r)
        start_scatter(jp, h, r)
        wait_scatter(((k - 1) // 2) % NPB, (k - 1) % 2, (r - 1) % NRB)
        if h == 0 and p + 2 < NPAIR:
            start_pload(p + 2, (jp + 2) % NPB)
        if k + 2 < NCHUNK:
            if h == 0:
                wait_pload((jp + 1) % NPB)
            start_gather((jp + 1) % NPB, h, (r + 2) % NRB)
    wait_scatter((NCHUNK - 1) // 2 % NPB, (NCHUNK - 1) % 2, (NCHUNK - 1) % NRB)
    plsc.subcore_barrier()

    pltpu.sync_copy(acc.at[pl.ds(s * ROWS_PER_TILE, ROWS_PER_TILE)],
                    out_hbm.at[c, pl.ds(s * ROWS_PER_TILE, ROWS_PER_TILE)])


def _make_sc_spmm():
    mesh = plsc.VectorSubcoreMesh(core_axis_name="c", subcore_axis_name="s")
    return pl.kernel(
        _sc_spmm_body,
        out_type=jax.ShapeDtypeStruct((NC, N_PAD, D), jnp.float32),
        mesh=mesh,
        scratch_types=(
            [pltpu.VMEM_SHARED((N_PAD, D), jnp.float32)]
            + [pltpu.VMEM((4, C), jnp.int32) for _ in range(NPB)]
            + [pltpu.VMEM((2, C), jnp.float32) for _ in range(NPB)]
            + [pltpu.VMEM((C, D), jnp.float32) for _ in range(NRB)]
            + [pltpu.SemaphoreType.DMA for _ in range(NPB + 2 * NRB)]
        ),
    )


_TC_ROWS = 2000
_NU_BLK = N_USERS // _TC_ROWS
_NI_BLK = N_ENTITIES // _TC_ROWS


def _l2n(x):
    nrm = jnp.sqrt(jnp.sum(x * x, axis=1, keepdims=True))
    return x / jnp.maximum(nrm, EPS)


def _dense(ego, parts, w):
    bi = ego * (parts[0] + parts[1])
    h = jnp.dot(bi, w, preferred_element_type=jnp.float32)
    return jnp.where(h > 0, h, h * 0.2)


def _tc_h_body(ego_ref, parts_ref, w_ref, h_ref):
    h_ref[...] = _dense(ego_ref[...], parts_ref, w_ref[...])


def _tc_hn_body(ego_ref, parts_ref, w_ref, h_ref, n_ref):
    ego = ego_ref[...]
    h_ref[...] = _dense(ego, parts_ref, w_ref[...])
    n_ref[...] = _l2n(ego)


_layer_in_specs = [
    pl.BlockSpec((_TC_ROWS, D), lambda i: (i, 0)),
    pl.BlockSpec((NC, _TC_ROWS, D), lambda i: (0, i, 0)),
    pl.BlockSpec((D, D), lambda i: (0, 0)),
]
_row_out_spec = pl.BlockSpec((_TC_ROWS, D), lambda i: (i, 0))

_tc_h = pl.pallas_call(
    _tc_h_body,
    grid=(N_NODES // _TC_ROWS,),
    in_specs=_layer_in_specs,
    out_specs=_row_out_spec,
    out_shape=jax.ShapeDtypeStruct((N_NODES, D), jnp.float32),
)

_tc_hn = pl.pallas_call(
    _tc_hn_body,
    grid=(N_NODES // _TC_ROWS,),
    in_specs=_layer_in_specs,
    out_specs=[_row_out_spec, _row_out_spec],
    out_shape=[
        jax.ShapeDtypeStruct((N_NODES, D), jnp.float32),
        jax.ShapeDtypeStruct((N_NODES, D), jnp.float32),
    ],
)


def _asm_body(e_ref, n1_ref, n2_ref, h3_ref, u_ref, i_ref):
    i = pl.program_id(0)
    n3 = _l2n(h3_ref[...])
    cols = (e_ref[...], n1_ref[...], n2_ref[...], n3)

    @pl.when(i == 0)
    def _():
        for t in range(4):
            u_ref[:, pl.ds(t * D, D)] = cols[t]

    @pl.when(i > 0)
    def _():
        for t in range(4):
            i_ref[:, pl.ds(t * D, D)] = cols[t]


_asm = pl.pallas_call(
    _asm_body,
    grid=(N_NODES // _TC_ROWS,),
    in_specs=[pl.BlockSpec((_TC_ROWS, D), lambda i: (i, 0))] * 4,
    out_specs=[
        pl.BlockSpec((N_USERS, 4 * D), lambda i: (0, 0)),
        pl.BlockSpec((_TC_ROWS, 4 * D), lambda i: (jnp.maximum(i - 1, 0), 0)),
    ],
    out_shape=[
        jax.ShapeDtypeStruct((N_USERS, 4 * D), jnp.float32),
        jax.ShapeDtypeStruct((N_ENTITIES, 4 * D), jnp.float32),
    ],
)


def kernel(user_embed, entity_embed, W0, W1, W2, edge_index, edge_vals):
    ego = jnp.concatenate([user_embed, entity_embed], axis=0)

    pad = E_PAD - N_EDGES
    fill = (jnp.arange(pad, dtype=jnp.int32) * 37) % N_NODES
    dst = jnp.concatenate([edge_index[0], fill]).reshape(NW, NCHUNK, C)
    src = jnp.concatenate([edge_index[1], fill]).reshape(NW, NCHUNK, C)
    val = jnp.concatenate(
        [edge_vals, jnp.zeros((pad,), jnp.float32)]).reshape(NW, NCHUNK, C)
    srcp = src.reshape(NW, NPAIR, 2, C)
    dstp = dst.reshape(NW, NPAIR, 2, C)
    edges = jnp.stack(
        [srcp[:, :, 0], dstp[:, :, 0], srcp[:, :, 1], dstp[:, :, 1]],
        axis=2)
    vals = val.reshape(NW, NPAIR, 2, C)

    sc_spmm = _make_sc_spmm()

    h1 = _tc_h(ego, sc_spmm(ego, edges, vals), W0)
    h2, n1 = _tc_hn(h1, sc_spmm(h1, edges, vals), W1)
    h3, n2 = _tc_hn(h2, sc_spmm(h2, edges, vals), W2)
    return _asm(ego, n1, n2, h3)
